# baseline (speedup 1.0000x reference)
import functools

import jax
import jax.numpy as jnp
from jax import lax
from jax.experimental import pallas as pl
from jax.experimental.pallas import tpu as pltpu
from jax.experimental.pallas import tpu_sc as plsc

N_NODES = 10000
FEATS = 128
HF = FEATS // 2
N_EDGES = 320000

NC = 2
NS = 16
NW = NC * NS
EPW = N_EDGES // NW
CHUNK = 80
EPWP = 10240
NCH = EPWP // CHUNK
NBUF = 4
NGRP = NCH // NBUF
NP = 10240
RPT = NP // NS
ZROWS = 128

_MM_BLOCK = 1000


def _mm_body(x_ref, w_ref, o_ref):
    o_ref[0] = lax.dot_general(
        x_ref[...], w_ref[0], (((1,), (1,)), ((), ())),
        preferred_element_type=jnp.float32)


def _mm_add_body(x_ref, w_ref, p_ref, o_ref):
    acc = lax.dot_general(
        x_ref[...], w_ref[...], (((1,), (1,)), ((), ())),
        preferred_element_type=jnp.float32)
    neigh = jnp.concatenate(
        [p_ref[0, 0] + p_ref[1, 0], p_ref[0, 1] + p_ref[1, 1]], axis=-1)
    o_ref[...] = acc + neigh


_mm = pl.pallas_call(
    _mm_body,
    grid=(2, N_NODES // _MM_BLOCK),
    in_specs=[
        pl.BlockSpec((_MM_BLOCK, FEATS), lambda h, i: (i, 0)),
        pl.BlockSpec((1, HF, FEATS), lambda h, i: (h, 0, 0)),
    ],
    out_specs=pl.BlockSpec((1, _MM_BLOCK, HF), lambda h, i: (h, i, 0)),
    out_shape=jax.ShapeDtypeStruct((2, N_NODES, HF), jnp.float32),
)

_mm_add = pl.pallas_call(
    _mm_add_body,
    grid=(N_NODES // _MM_BLOCK,),
    in_specs=[
        pl.BlockSpec((_MM_BLOCK, FEATS), lambda i: (i, 0)),
        pl.BlockSpec((FEATS, FEATS), lambda i: (0, 0)),
        pl.BlockSpec((NC, 2, _MM_BLOCK, HF), lambda i: (0, 0, i, 0)),
    ],
    out_specs=pl.BlockSpec((_MM_BLOCK, FEATS), lambda i: (i, 0)),
    out_shape=jax.ShapeDtypeStruct((N_NODES, FEATS), jnp.float32),
)


@functools.partial(
    pl.kernel,
    out_type=jax.ShapeDtypeStruct((NC, 2, NP, HF), jnp.float32),
    mesh=plsc.VectorSubcoreMesh(core_axis_name="c", subcore_axis_name="s"),
    compiler_params=pltpu.CompilerParams(use_tc_tiling_on_sc=False),
    scratch_types=[
        pltpu.VMEM((NCH, CHUNK), jnp.int32),
        pltpu.VMEM((NCH, CHUNK), jnp.int32),
        pltpu.VMEM((NBUF, CHUNK, HF), jnp.float32),
        pltpu.VMEM((ZROWS, HF), jnp.float32),
        pltpu.VMEM_SHARED((NP, HF), jnp.float32),
        [pltpu.SemaphoreType.DMA] * NBUF,
        [pltpu.SemaphoreType.DMA] * NBUF,
    ],
)
def _sc_spmm(fn_hbm, src_hbm, dst_hbm, out_hbm,
             src_v, dst_v, rows_v, zbuf_v, acc_sh, gsem, ssem):
    c = lax.axis_index("c")
    s = lax.axis_index("s")
    wid = c * NS + s

    zero = jnp.zeros((16,), jnp.float32)

    def _zero_body(t, carry):
        zbuf_v[t // (HF // 16), pl.ds((t % (HF // 16)) * 16, 16)] = zero
        return carry

    lax.fori_loop(0, ZROWS * (HF // 16), _zero_body, 0)

    pltpu.sync_copy(src_hbm.at[wid], src_v)
    pltpu.sync_copy(dst_hbm.at[wid], dst_v)

    for h in range(2):
        for k in range(RPT // ZROWS):
            pltpu.sync_copy(zbuf_v, acc_sh.at[pl.ds(s * RPT + k * ZROWS, ZROWS)])
        plsc.subcore_barrier()

        def _group_body(g, carry):
            j0 = g * NBUF
            gd = [pltpu.async_copy(fn_hbm.at[h].at[src_v.at[j0 + b]],
                                   rows_v.at[b], gsem[b])
                  for b in range(NBUF)]
            sd = []
            for b in range(NBUF):
                gd[b].wait()
                sd.append(pltpu.async_copy(rows_v.at[b],
                                           acc_sh.at[dst_v.at[j0 + b]],
                                           ssem[b], add=True))
            for b in range(NBUF):
                sd[b].wait()
            return carry

        lax.fori_loop(0, NGRP, _group_body, 0)
        plsc.subcore_barrier()

        for k in range(RPT // ZROWS):
            base = s * RPT + k * ZROWS
            pltpu.sync_copy(acc_sh.at[pl.ds(base, ZROWS)],
                            out_hbm.at[c, h, pl.ds(base, ZROWS)])
        plsc.subcore_barrier()


def kernel(feat, edge_index, W_self, W_neigh):
    ei = edge_index.astype(jnp.int32)
    pad = EPWP - EPW
    src = jnp.pad(ei[0].reshape(NW, EPW), ((0, 0), (0, pad))
                  ).reshape(NW, NCH, CHUNK)
    pad_dst = N_NODES + (jnp.arange(pad, dtype=jnp.int32) % (NP - N_NODES))
    dst = jnp.concatenate(
        [ei[1].reshape(NW, EPW), jnp.broadcast_to(pad_dst, (NW, pad))],
        axis=1).reshape(NW, NCH, CHUNK)
    fn = _mm(feat, W_neigh.reshape(2, HF, FEATS))
    parts = _sc_spmm(fn, src, dst)
    return _mm_add(feat, W_self, parts)


# speedup vs baseline: 2.1444x; 2.1444x over previous
import functools

import jax
import jax.numpy as jnp
from jax import lax
from jax.experimental import pallas as pl
from jax.experimental.pallas import tpu as pltpu
from jax.experimental.pallas import tpu_sc as plsc

N_NODES = 10000
FEATS = 128
HF = FEATS // 2
N_EDGES = 320000

NC = 2
NS = 16
NW = NC * NS
EPW = N_EDGES // NW
CHUNK = 80
EPWP = 10240
NCH = EPWP // CHUNK
NBUF = 4
NGRP = NCH // NBUF
NP = 10240
RPT = NP // NS
ZROWS = 128

_MM_BLOCK = 1000


def _mm_body(x_ref, w_ref, o_ref):
    o_ref[0] = lax.dot_general(
        x_ref[...], w_ref[0], (((1,), (1,)), ((), ())),
        preferred_element_type=jnp.float32)


def _mm_add_body(x_ref, w_ref, p_ref, o_ref):
    acc = lax.dot_general(
        x_ref[...], w_ref[...], (((1,), (1,)), ((), ())),
        preferred_element_type=jnp.float32)
    neigh = jnp.concatenate(
        [p_ref[0, 0] + p_ref[1, 0], p_ref[0, 1] + p_ref[1, 1]], axis=-1)
    o_ref[...] = acc + neigh


_mm = pl.pallas_call(
    _mm_body,
    grid=(2, N_NODES // _MM_BLOCK),
    in_specs=[
        pl.BlockSpec((_MM_BLOCK, FEATS), lambda h, i: (i, 0)),
        pl.BlockSpec((1, HF, FEATS), lambda h, i: (h, 0, 0)),
    ],
    out_specs=pl.BlockSpec((1, _MM_BLOCK, HF), lambda h, i: (h, i, 0)),
    out_shape=jax.ShapeDtypeStruct((2, N_NODES, HF), jnp.float32),
)

_mm_add = pl.pallas_call(
    _mm_add_body,
    grid=(N_NODES // _MM_BLOCK,),
    in_specs=[
        pl.BlockSpec((_MM_BLOCK, FEATS), lambda i: (i, 0)),
        pl.BlockSpec((FEATS, FEATS), lambda i: (0, 0)),
        pl.BlockSpec((NC, 2, _MM_BLOCK, HF), lambda i: (0, 0, i, 0)),
    ],
    out_specs=pl.BlockSpec((_MM_BLOCK, FEATS), lambda i: (i, 0)),
    out_shape=jax.ShapeDtypeStruct((N_NODES, FEATS), jnp.float32),
)


@functools.partial(
    pl.kernel,
    out_type=jax.ShapeDtypeStruct((NC, 2, NP, HF), jnp.float32),
    mesh=plsc.VectorSubcoreMesh(core_axis_name="c", subcore_axis_name="s"),
    compiler_params=pltpu.CompilerParams(use_tc_tiling_on_sc=False),
    scratch_types=[
        pltpu.VMEM((NCH, CHUNK), jnp.int32),
        pltpu.VMEM((NCH, CHUNK), jnp.int32),
        pltpu.VMEM((NBUF, CHUNK, HF), jnp.float32),
        pltpu.VMEM((ZROWS, HF), jnp.float32),
        pltpu.VMEM_SHARED((NP, HF), jnp.float32),
        [pltpu.SemaphoreType.DMA] * NBUF,
        [pltpu.SemaphoreType.DMA] * NBUF,
    ],
)
def _sc_spmm(fn_hbm, src_hbm, dst_hbm, out_hbm,
             src_v, dst_v, rows_v, zbuf_v, acc_sh, gsem, ssem):
    c = lax.axis_index("c")
    s = lax.axis_index("s")
    wid = c * NS + s

    zero = jnp.zeros((16,), jnp.float32)

    def _zero_body(t, carry):
        zbuf_v[t // (HF // 16), pl.ds((t % (HF // 16)) * 16, 16)] = zero
        return carry

    lax.fori_loop(0, ZROWS * (HF // 16), _zero_body, 0)

    pltpu.sync_copy(src_hbm.at[wid], src_v)
    pltpu.sync_copy(dst_hbm.at[wid], dst_v)

    for h in range(2):
        for k in range(RPT // ZROWS):
            pltpu.sync_copy(zbuf_v, acc_sh.at[pl.ds(s * RPT + k * ZROWS, ZROWS)])
        plsc.subcore_barrier()

        def _group_body(g, carry):
            j0 = g * NBUF
            gd = [pltpu.async_copy(fn_hbm.at[h].at[src_v.at[j0 + b]],
                                   rows_v.at[b], gsem[b])
                  for b in range(NBUF)]
            sd = []
            for b in range(NBUF):
                gd[b].wait()
                sd.append(pltpu.async_copy(rows_v.at[b],
                                           acc_sh.at[dst_v.at[j0 + b]],
                                           ssem[b], add=True))
            for b in range(NBUF):
                sd[b].wait()
            return carry

        lax.fori_loop(0, NGRP, _group_body, 0)
        plsc.subcore_barrier()

        for k in range(RPT // ZROWS):
            base = s * RPT + k * ZROWS
            pltpu.sync_copy(acc_sh.at[pl.ds(base, ZROWS)],
                            out_hbm.at[c, h, pl.ds(base, ZROWS)])
        plsc.subcore_barrier()


def kernel(feat, edge_index, W_self, W_neigh):
    ei = edge_index.astype(jnp.int32)
    pad = EPWP - EPW
    pad_src = (jnp.arange(pad, dtype=jnp.int32) * 37) % N_NODES
    src = jnp.concatenate(
        [ei[0].reshape(NW, EPW), jnp.broadcast_to(pad_src, (NW, pad))],
        axis=1).reshape(NW, NCH, CHUNK)
    pad_dst = N_NODES + (jnp.arange(pad, dtype=jnp.int32) % (NP - N_NODES))
    dst = jnp.concatenate(
        [ei[1].reshape(NW, EPW), jnp.broadcast_to(pad_dst, (NW, pad))],
        axis=1).reshape(NW, NCH, CHUNK)
    fn = _mm(feat, W_neigh.reshape(2, HF, FEATS))
    parts = _sc_spmm(fn, src, dst)
    return _mm_add(feat, W_self, parts)


# speedup vs baseline: 2.2220x; 1.0362x over previous
import functools

import jax
import jax.numpy as jnp
from jax import lax
from jax.experimental import pallas as pl
from jax.experimental.pallas import tpu as pltpu
from jax.experimental.pallas import tpu_sc as plsc

N_NODES = 10000
FEATS = 128
HF = FEATS // 2
N_EDGES = 320000

NC = 2
NS = 16
NW = NC * NS
EPW = N_EDGES // NW
CHUNK = 128
EPWP = 10240
NCH = EPWP // CHUNK
NBUF = 4
NGRP = NCH // NBUF
NP = 10240
RPT = NP // NS
ZROWS = 128

_MM_BLOCK = 1000


def _mm_body(x_ref, w_ref, o_ref):
    o_ref[0] = lax.dot_general(
        x_ref[...], w_ref[0], (((1,), (1,)), ((), ())),
        preferred_element_type=jnp.float32)


def _mm_add_body(x_ref, w_ref, p_ref, o_ref):
    acc = lax.dot_general(
        x_ref[...], w_ref[...], (((1,), (1,)), ((), ())),
        preferred_element_type=jnp.float32)
    neigh = jnp.concatenate(
        [p_ref[0, 0] + p_ref[1, 0], p_ref[0, 1] + p_ref[1, 1]], axis=-1)
    o_ref[...] = acc + neigh


_mm = pl.pallas_call(
    _mm_body,
    grid=(2, N_NODES // _MM_BLOCK),
    in_specs=[
        pl.BlockSpec((_MM_BLOCK, FEATS), lambda h, i: (i, 0)),
        pl.BlockSpec((1, HF, FEATS), lambda h, i: (h, 0, 0)),
    ],
    out_specs=pl.BlockSpec((1, _MM_BLOCK, HF), lambda h, i: (h, i, 0)),
    out_shape=jax.ShapeDtypeStruct((2, N_NODES, HF), jnp.float32),
)

_mm_add = pl.pallas_call(
    _mm_add_body,
    grid=(N_NODES // _MM_BLOCK,),
    in_specs=[
        pl.BlockSpec((_MM_BLOCK, FEATS), lambda i: (i, 0)),
        pl.BlockSpec((FEATS, FEATS), lambda i: (0, 0)),
        pl.BlockSpec((NC, 2, _MM_BLOCK, HF), lambda i: (0, 0, i, 0)),
    ],
    out_specs=pl.BlockSpec((_MM_BLOCK, FEATS), lambda i: (i, 0)),
    out_shape=jax.ShapeDtypeStruct((N_NODES, FEATS), jnp.float32),
)


@functools.partial(
    pl.kernel,
    out_type=jax.ShapeDtypeStruct((NC, 2, NP, HF), jnp.float32),
    mesh=plsc.VectorSubcoreMesh(core_axis_name="c", subcore_axis_name="s"),
    compiler_params=pltpu.CompilerParams(use_tc_tiling_on_sc=False),
    scratch_types=[
        pltpu.VMEM((NCH, CHUNK), jnp.int32),
        pltpu.VMEM((NCH, CHUNK), jnp.int32),
        pltpu.VMEM((NBUF, CHUNK, HF), jnp.float32),
        pltpu.VMEM((ZROWS, HF), jnp.float32),
        pltpu.VMEM_SHARED((NP, HF), jnp.float32),
        [pltpu.SemaphoreType.DMA] * NBUF,
        [pltpu.SemaphoreType.DMA] * NBUF,
    ],
)
def _sc_spmm(fn_hbm, src_hbm, dst_hbm, out_hbm,
             src_v, dst_v, rows_v, zbuf_v, acc_sh, gsem, ssem):
    c = lax.axis_index("c")
    s = lax.axis_index("s")
    wid = c * NS + s

    zero = jnp.zeros((16,), jnp.float32)

    def _zero_body(t, carry):
        zbuf_v[t // (HF // 16), pl.ds((t % (HF // 16)) * 16, 16)] = zero
        return carry

    lax.fori_loop(0, ZROWS * (HF // 16), _zero_body, 0)

    pltpu.sync_copy(src_hbm.at[wid], src_v)
    pltpu.sync_copy(dst_hbm.at[wid], dst_v)

    for h in range(2):
        for k in range(RPT // ZROWS):
            pltpu.sync_copy(zbuf_v, acc_sh.at[pl.ds(s * RPT + k * ZROWS, ZROWS)])
        plsc.subcore_barrier()

        def _group_body(g, carry):
            j0 = g * NBUF
            gd = [pltpu.async_copy(fn_hbm.at[h].at[src_v.at[j0 + b]],
                                   rows_v.at[b], gsem[b])
                  for b in range(NBUF)]
            sd = []
            for b in range(NBUF):
                gd[b].wait()
                sd.append(pltpu.async_copy(rows_v.at[b],
                                           acc_sh.at[dst_v.at[j0 + b]],
                                           ssem[b], add=True))
            for b in range(NBUF):
                sd[b].wait()
            return carry

        lax.fori_loop(0, NGRP, _group_body, 0)
        plsc.subcore_barrier()

        for k in range(RPT // ZROWS):
            base = s * RPT + k * ZROWS
            pltpu.sync_copy(acc_sh.at[pl.ds(base, ZROWS)],
                            out_hbm.at[c, h, pl.ds(base, ZROWS)])
        plsc.subcore_barrier()


def kernel(feat, edge_index, W_self, W_neigh):
    ei = edge_index.astype(jnp.int32)
    pad = EPWP - EPW
    pad_src = (jnp.arange(pad, dtype=jnp.int32) * 37) % N_NODES
    src = jnp.concatenate(
        [ei[0].reshape(NW, EPW), jnp.broadcast_to(pad_src, (NW, pad))],
        axis=1).reshape(NW, NCH, CHUNK)
    pad_dst = N_NODES + (jnp.arange(pad, dtype=jnp.int32) % (NP - N_NODES))
    dst = jnp.concatenate(
        [ei[1].reshape(NW, EPW), jnp.broadcast_to(pad_dst, (NW, pad))],
        axis=1).reshape(NW, NCH, CHUNK)
    fn = _mm(feat, W_neigh.reshape(2, HF, FEATS))
    parts = _sc_spmm(fn, src, dst)
    return _mm_add(feat, W_self, parts)


# speedup vs baseline: 2.2766x; 1.0246x over previous
import functools

import jax
import jax.numpy as jnp
from jax import lax
from jax.experimental import pallas as pl
from jax.experimental.pallas import tpu as pltpu
from jax.experimental.pallas import tpu_sc as plsc

N_NODES = 10000
FEATS = 128
HF = FEATS // 2
N_EDGES = 320000

NC = 2
NS = 16
NW = NC * NS
EPW = N_EDGES // NW
CHUNK = 128
EPWP = 10240
NCH = EPWP // CHUNK
NBUF = 5
NGRP = NCH // NBUF
NP = 10240
RPT = NP // NS
ZROWS = 128

_MM_BLOCK = 1000


def _mm_body(x_ref, w_ref, o_ref):
    o_ref[0] = lax.dot_general(
        x_ref[...], w_ref[0], (((1,), (1,)), ((), ())),
        preferred_element_type=jnp.float32)


def _mm_add_body(x_ref, w_ref, p_ref, o_ref):
    acc = lax.dot_general(
        x_ref[...], w_ref[...], (((1,), (1,)), ((), ())),
        preferred_element_type=jnp.float32)
    neigh = jnp.concatenate(
        [p_ref[0, 0] + p_ref[1, 0], p_ref[0, 1] + p_ref[1, 1]], axis=-1)
    o_ref[...] = acc + neigh


_mm = pl.pallas_call(
    _mm_body,
    grid=(2, N_NODES // _MM_BLOCK),
    in_specs=[
        pl.BlockSpec((_MM_BLOCK, FEATS), lambda h, i: (i, 0)),
        pl.BlockSpec((1, HF, FEATS), lambda h, i: (h, 0, 0)),
    ],
    out_specs=pl.BlockSpec((1, _MM_BLOCK, HF), lambda h, i: (h, i, 0)),
    out_shape=jax.ShapeDtypeStruct((2, N_NODES, HF), jnp.float32),
)

_mm_add = pl.pallas_call(
    _mm_add_body,
    grid=(N_NODES // _MM_BLOCK,),
    in_specs=[
        pl.BlockSpec((_MM_BLOCK, FEATS), lambda i: (i, 0)),
        pl.BlockSpec((FEATS, FEATS), lambda i: (0, 0)),
        pl.BlockSpec((NC, 2, _MM_BLOCK, HF), lambda i: (0, 0, i, 0)),
    ],
    out_specs=pl.BlockSpec((_MM_BLOCK, FEATS), lambda i: (i, 0)),
    out_shape=jax.ShapeDtypeStruct((N_NODES, FEATS), jnp.float32),
)


@functools.partial(
    pl.kernel,
    out_type=jax.ShapeDtypeStruct((NC, 2, NP, HF), jnp.float32),
    mesh=plsc.VectorSubcoreMesh(core_axis_name="c", subcore_axis_name="s"),
    compiler_params=pltpu.CompilerParams(use_tc_tiling_on_sc=False),
    scratch_types=[
        pltpu.VMEM((NCH, CHUNK), jnp.int32),
        pltpu.VMEM((NCH, CHUNK), jnp.int32),
        pltpu.VMEM((NBUF, CHUNK, HF), jnp.float32),
        pltpu.VMEM((ZROWS, HF), jnp.float32),
        pltpu.VMEM_SHARED((NP, HF), jnp.float32),
        [pltpu.SemaphoreType.DMA] * NBUF,
        [pltpu.SemaphoreType.DMA] * NBUF,
    ],
)
def _sc_spmm(fn_hbm, src_hbm, dst_hbm, out_hbm,
             src_v, dst_v, rows_v, zbuf_v, acc_sh, gsem, ssem):
    c = lax.axis_index("c")
    s = lax.axis_index("s")
    wid = c * NS + s

    zero = jnp.zeros((16,), jnp.float32)

    def _zero_body(t, carry):
        zbuf_v[t // (HF // 16), pl.ds((t % (HF // 16)) * 16, 16)] = zero
        return carry

    lax.fori_loop(0, ZROWS * (HF // 16), _zero_body, 0)

    pltpu.sync_copy(src_hbm.at[wid], src_v)
    pltpu.sync_copy(dst_hbm.at[wid], dst_v)

    for h in range(2):
        for k in range(RPT // ZROWS):
            pltpu.sync_copy(zbuf_v, acc_sh.at[pl.ds(s * RPT + k * ZROWS, ZROWS)])
        plsc.subcore_barrier()

        def _group_body(g, carry):
            j0 = g * NBUF
            gd = [pltpu.async_copy(fn_hbm.at[h].at[src_v.at[j0 + b]],
                                   rows_v.at[b], gsem[b])
                  for b in range(NBUF)]
            sd = []
            for b in range(NBUF):
                gd[b].wait()
                sd.append(pltpu.async_copy(rows_v.at[b],
                                           acc_sh.at[dst_v.at[j0 + b]],
                                           ssem[b], add=True))
            for b in range(NBUF):
                sd[b].wait()
            return carry

        lax.fori_loop(0, NGRP, _group_body, 0)
        plsc.subcore_barrier()

        for k in range(RPT // ZROWS):
            base = s * RPT + k * ZROWS
            pltpu.sync_copy(acc_sh.at[pl.ds(base, ZROWS)],
                            out_hbm.at[c, h, pl.ds(base, ZROWS)])
        plsc.subcore_barrier()


def kernel(feat, edge_index, W_self, W_neigh):
    ei = edge_index.astype(jnp.int32)
    pad = EPWP - EPW
    pad_src = (jnp.arange(pad, dtype=jnp.int32) * 37) % N_NODES
    src = jnp.concatenate(
        [ei[0].reshape(NW, EPW), jnp.broadcast_to(pad_src, (NW, pad))],
        axis=1).reshape(NW, NCH, CHUNK)
    pad_dst = N_NODES + (jnp.arange(pad, dtype=jnp.int32) % (NP - N_NODES))
    dst = jnp.concatenate(
        [ei[1].reshape(NW, EPW), jnp.broadcast_to(pad_dst, (NW, pad))],
        axis=1).reshape(NW, NCH, CHUNK)
    fn = _mm(feat, W_neigh.reshape(2, HF, FEATS))
    parts = _sc_spmm(fn, src, dst)
    return _mm_add(feat, W_self, parts)


# speedup vs baseline: 2.2793x; 1.0012x over previous
import functools

import jax
import jax.numpy as jnp
from jax import lax
from jax.experimental import pallas as pl
from jax.experimental.pallas import tpu as pltpu
from jax.experimental.pallas import tpu_sc as plsc

N_NODES = 10000
FEATS = 128
HF = FEATS // 2
N_EDGES = 320000

NC = 2
NS = 16
NW = NC * NS
EPW = N_EDGES // NW
CHUNK = 80
EPWP = 10080
NCH = EPWP // CHUNK
NBUF = 6
NGRP = NCH // NBUF
NP = 10240
RPT = NP // NS
ZROWS = 128

_MM_BLOCK = 1000


def _mm_body(x_ref, w_ref, o_ref):
    o_ref[0] = lax.dot_general(
        x_ref[...], w_ref[0], (((1,), (1,)), ((), ())),
        preferred_element_type=jnp.float32)


def _mm_add_body(x_ref, w_ref, p_ref, o_ref):
    acc = lax.dot_general(
        x_ref[...], w_ref[...], (((1,), (1,)), ((), ())),
        preferred_element_type=jnp.float32)
    neigh = jnp.concatenate(
        [p_ref[0, 0] + p_ref[1, 0], p_ref[0, 1] + p_ref[1, 1]], axis=-1)
    o_ref[...] = acc + neigh


_mm = pl.pallas_call(
    _mm_body,
    grid=(2, N_NODES // _MM_BLOCK),
    in_specs=[
        pl.BlockSpec((_MM_BLOCK, FEATS), lambda h, i: (i, 0)),
        pl.BlockSpec((1, HF, FEATS), lambda h, i: (h, 0, 0)),
    ],
    out_specs=pl.BlockSpec((1, _MM_BLOCK, HF), lambda h, i: (h, i, 0)),
    out_shape=jax.ShapeDtypeStruct((2, N_NODES, HF), jnp.float32),
)

_mm_add = pl.pallas_call(
    _mm_add_body,
    grid=(N_NODES // _MM_BLOCK,),
    in_specs=[
        pl.BlockSpec((_MM_BLOCK, FEATS), lambda i: (i, 0)),
        pl.BlockSpec((FEATS, FEATS), lambda i: (0, 0)),
        pl.BlockSpec((NC, 2, _MM_BLOCK, HF), lambda i: (0, 0, i, 0)),
    ],
    out_specs=pl.BlockSpec((_MM_BLOCK, FEATS), lambda i: (i, 0)),
    out_shape=jax.ShapeDtypeStruct((N_NODES, FEATS), jnp.float32),
)


@functools.partial(
    pl.kernel,
    out_type=jax.ShapeDtypeStruct((NC, 2, NP, HF), jnp.float32),
    mesh=plsc.VectorSubcoreMesh(core_axis_name="c", subcore_axis_name="s"),
    compiler_params=pltpu.CompilerParams(use_tc_tiling_on_sc=False),
    scratch_types=[
        pltpu.VMEM((NCH, CHUNK), jnp.int32),
        pltpu.VMEM((NCH, CHUNK), jnp.int32),
        pltpu.VMEM((NBUF, CHUNK, HF), jnp.float32),
        pltpu.VMEM((ZROWS, HF), jnp.float32),
        pltpu.VMEM_SHARED((NP, HF), jnp.float32),
        [pltpu.SemaphoreType.DMA] * NBUF,
        [pltpu.SemaphoreType.DMA] * NBUF,
    ],
)
def _sc_spmm(fn_hbm, src_hbm, dst_hbm, out_hbm,
             src_v, dst_v, rows_v, zbuf_v, acc_sh, gsem, ssem):
    c = lax.axis_index("c")
    s = lax.axis_index("s")
    wid = c * NS + s

    zero = jnp.zeros((16,), jnp.float32)

    def _zero_body(t, carry):
        zbuf_v[t // (HF // 16), pl.ds((t % (HF // 16)) * 16, 16)] = zero
        return carry

    lax.fori_loop(0, ZROWS * (HF // 16), _zero_body, 0)

    pltpu.sync_copy(src_hbm.at[wid], src_v)
    pltpu.sync_copy(dst_hbm.at[wid], dst_v)

    for h in range(2):
        for k in range(RPT // ZROWS):
            pltpu.sync_copy(zbuf_v, acc_sh.at[pl.ds(s * RPT + k * ZROWS, ZROWS)])
        plsc.subcore_barrier()

        def _group_body(g, carry):
            j0 = g * NBUF
            gd = [pltpu.async_copy(fn_hbm.at[h].at[src_v.at[j0 + b]],
                                   rows_v.at[b], gsem[b])
                  for b in range(NBUF)]
            sd = []
            for b in range(NBUF):
                gd[b].wait()
                sd.append(pltpu.async_copy(rows_v.at[b],
                                           acc_sh.at[dst_v.at[j0 + b]],
                                           ssem[b], add=True))
            for b in range(NBUF):
                sd[b].wait()
            return carry

        lax.fori_loop(0, NGRP, _group_body, 0)
        plsc.subcore_barrier()

        for k in range(RPT // ZROWS):
            base = s * RPT + k * ZROWS
            pltpu.sync_copy(acc_sh.at[pl.ds(base, ZROWS)],
                            out_hbm.at[c, h, pl.ds(base, ZROWS)])
        plsc.subcore_barrier()


def kernel(feat, edge_index, W_self, W_neigh):
    ei = edge_index.astype(jnp.int32)
    pad = EPWP - EPW
    pad_src = (jnp.arange(pad, dtype=jnp.int32) * 37) % N_NODES
    src = jnp.concatenate(
        [ei[0].reshape(NW, EPW), jnp.broadcast_to(pad_src, (NW, pad))],
        axis=1).reshape(NW, NCH, CHUNK)
    pad_dst = N_NODES + (jnp.arange(pad, dtype=jnp.int32) % (NP - N_NODES))
    dst = jnp.concatenate(
        [ei[1].reshape(NW, EPW), jnp.broadcast_to(pad_dst, (NW, pad))],
        axis=1).reshape(NW, NCH, CHUNK)
    fn = _mm(feat, W_neigh.reshape(2, HF, FEATS))
    parts = _sc_spmm(fn, src, dst)
    return _mm_add(feat, W_self, parts)


# speedup vs baseline: 2.3459x; 1.0292x over previous
import functools

import jax
import jax.numpy as jnp
from jax import lax
from jax.experimental import pallas as pl
from jax.experimental.pallas import tpu as pltpu
from jax.experimental.pallas import tpu_sc as plsc

N_NODES = 10000
FEATS = 128
HF = FEATS // 2
N_EDGES = 320000

NC = 2
NS = 16
NW = NC * NS
EPW = N_EDGES // NW
CHUNK = 80
EPWP = 10240
NCH = EPWP // CHUNK
NBUF = 8
NGRP = NCH // NBUF
NP = 10240
RPT = NP // NS
ZROWS = 128

_MM_BLOCK = 1000


def _mm_body(x_ref, w_ref, o_ref):
    o_ref[0] = lax.dot_general(
        x_ref[...], w_ref[0], (((1,), (1,)), ((), ())),
        preferred_element_type=jnp.float32)


def _mm_add_body(x_ref, w_ref, p_ref, o_ref):
    acc = lax.dot_general(
        x_ref[...], w_ref[...], (((1,), (1,)), ((), ())),
        preferred_element_type=jnp.float32)
    neigh = jnp.concatenate(
        [p_ref[0, 0] + p_ref[1, 0], p_ref[0, 1] + p_ref[1, 1]], axis=-1)
    o_ref[...] = acc + neigh


_mm = pl.pallas_call(
    _mm_body,
    grid=(2, N_NODES // _MM_BLOCK),
    in_specs=[
        pl.BlockSpec((_MM_BLOCK, FEATS), lambda h, i: (i, 0)),
        pl.BlockSpec((1, HF, FEATS), lambda h, i: (h, 0, 0)),
    ],
    out_specs=pl.BlockSpec((1, _MM_BLOCK, HF), lambda h, i: (h, i, 0)),
    out_shape=jax.ShapeDtypeStruct((2, N_NODES, HF), jnp.float32),
)

_mm_add = pl.pallas_call(
    _mm_add_body,
    grid=(N_NODES // _MM_BLOCK,),
    in_specs=[
        pl.BlockSpec((_MM_BLOCK, FEATS), lambda i: (i, 0)),
        pl.BlockSpec((FEATS, FEATS), lambda i: (0, 0)),
        pl.BlockSpec((NC, 2, _MM_BLOCK, HF), lambda i: (0, 0, i, 0)),
    ],
    out_specs=pl.BlockSpec((_MM_BLOCK, FEATS), lambda i: (i, 0)),
    out_shape=jax.ShapeDtypeStruct((N_NODES, FEATS), jnp.float32),
)


@functools.partial(
    pl.kernel,
    out_type=jax.ShapeDtypeStruct((NC, 2, NP, HF), jnp.float32),
    mesh=plsc.VectorSubcoreMesh(core_axis_name="c", subcore_axis_name="s"),
    compiler_params=pltpu.CompilerParams(use_tc_tiling_on_sc=False),
    scratch_types=[
        pltpu.VMEM((NCH, CHUNK), jnp.int32),
        pltpu.VMEM((NCH, CHUNK), jnp.int32),
        pltpu.VMEM((NBUF, CHUNK, HF), jnp.float32),
        pltpu.VMEM((ZROWS, HF), jnp.float32),
        pltpu.VMEM_SHARED((NP, HF), jnp.float32),
        [pltpu.SemaphoreType.DMA] * NBUF,
        [pltpu.SemaphoreType.DMA] * NBUF,
    ],
)
def _sc_spmm(fn_hbm, src_hbm, dst_hbm, out_hbm,
             src_v, dst_v, rows_v, zbuf_v, acc_sh, gsem, ssem):
    c = lax.axis_index("c")
    s = lax.axis_index("s")
    wid = c * NS + s

    zero = jnp.zeros((16,), jnp.float32)

    def _zero_body(t, carry):
        zbuf_v[t // (HF // 16), pl.ds((t % (HF // 16)) * 16, 16)] = zero
        return carry

    lax.fori_loop(0, ZROWS * (HF // 16), _zero_body, 0)

    pltpu.sync_copy(src_hbm.at[wid], src_v)
    pltpu.sync_copy(dst_hbm.at[wid], dst_v)

    for h in range(2):
        for k in range(RPT // ZROWS):
            pltpu.sync_copy(zbuf_v, acc_sh.at[pl.ds(s * RPT + k * ZROWS, ZROWS)])
        plsc.subcore_barrier()

        def _group_body(g, carry):
            j0 = g * NBUF
            gd = [pltpu.async_copy(fn_hbm.at[h].at[src_v.at[j0 + b]],
                                   rows_v.at[b], gsem[b])
                  for b in range(NBUF)]
            sd = []
            for b in range(NBUF):
                gd[b].wait()
                sd.append(pltpu.async_copy(rows_v.at[b],
                                           acc_sh.at[dst_v.at[j0 + b]],
                                           ssem[b], add=True))
            for b in range(NBUF):
                sd[b].wait()
            return carry

        lax.fori_loop(0, NGRP, _group_body, 0)
        plsc.subcore_barrier()

        for k in range(RPT // ZROWS):
            base = s * RPT + k * ZROWS
            pltpu.sync_copy(acc_sh.at[pl.ds(base, ZROWS)],
                            out_hbm.at[c, h, pl.ds(base, ZROWS)])
        plsc.subcore_barrier()


def kernel(feat, edge_index, W_self, W_neigh):
    ei = edge_index.astype(jnp.int32)
    pad = EPWP - EPW
    pad_src = (jnp.arange(pad, dtype=jnp.int32) * 37) % N_NODES
    src = jnp.concatenate(
        [ei[0].reshape(NW, EPW), jnp.broadcast_to(pad_src, (NW, pad))],
        axis=1).reshape(NW, NCH, CHUNK)
    pad_dst = N_NODES + (jnp.arange(pad, dtype=jnp.int32) % (NP - N_NODES))
    dst = jnp.concatenate(
        [ei[1].reshape(NW, EPW), jnp.broadcast_to(pad_dst, (NW, pad))],
        axis=1).reshape(NW, NCH, CHUNK)
    fn = _mm(feat, W_neigh.reshape(2, HF, FEATS))
    parts = _sc_spmm(fn, src, dst)
    return _mm_add(feat, W_self, parts)


# speedup vs baseline: 2.4141x; 1.0291x over previous
import functools

import jax
import jax.numpy as jnp
from jax import lax
from jax.experimental import pallas as pl
from jax.experimental.pallas import tpu as pltpu
from jax.experimental.pallas import tpu_sc as plsc

N_NODES = 10000
FEATS = 128
HF = FEATS // 2
N_EDGES = 320000

NC = 2
NS = 16
NW = NC * NS
EPW = N_EDGES // NW
CHUNK = 64
EPWP = 10240
NCH = EPWP // CHUNK
NBUF = 10
NGRP = NCH // NBUF
NP = 10240
RPT = NP // NS
ZROWS = 128

_MM_BLOCK = 1000


def _mm_body(x_ref, w_ref, o_ref):
    o_ref[0] = lax.dot_general(
        x_ref[...], w_ref[0], (((1,), (1,)), ((), ())),
        preferred_element_type=jnp.float32)


def _mm_add_body(x_ref, w_ref, p_ref, o_ref):
    acc = lax.dot_general(
        x_ref[...], w_ref[...], (((1,), (1,)), ((), ())),
        preferred_element_type=jnp.float32)
    neigh = jnp.concatenate(
        [p_ref[0, 0] + p_ref[1, 0], p_ref[0, 1] + p_ref[1, 1]], axis=-1)
    o_ref[...] = acc + neigh


_mm = pl.pallas_call(
    _mm_body,
    grid=(2, N_NODES // _MM_BLOCK),
    in_specs=[
        pl.BlockSpec((_MM_BLOCK, FEATS), lambda h, i: (i, 0)),
        pl.BlockSpec((1, HF, FEATS), lambda h, i: (h, 0, 0)),
    ],
    out_specs=pl.BlockSpec((1, _MM_BLOCK, HF), lambda h, i: (h, i, 0)),
    out_shape=jax.ShapeDtypeStruct((2, N_NODES, HF), jnp.float32),
)

_mm_add = pl.pallas_call(
    _mm_add_body,
    grid=(N_NODES // _MM_BLOCK,),
    in_specs=[
        pl.BlockSpec((_MM_BLOCK, FEATS), lambda i: (i, 0)),
        pl.BlockSpec((FEATS, FEATS), lambda i: (0, 0)),
        pl.BlockSpec((NC, 2, _MM_BLOCK, HF), lambda i: (0, 0, i, 0)),
    ],
    out_specs=pl.BlockSpec((_MM_BLOCK, FEATS), lambda i: (i, 0)),
    out_shape=jax.ShapeDtypeStruct((N_NODES, FEATS), jnp.float32),
)


@functools.partial(
    pl.kernel,
    out_type=jax.ShapeDtypeStruct((NC, 2, NP, HF), jnp.float32),
    mesh=plsc.VectorSubcoreMesh(core_axis_name="c", subcore_axis_name="s"),
    compiler_params=pltpu.CompilerParams(use_tc_tiling_on_sc=False),
    scratch_types=[
        pltpu.VMEM((NCH, CHUNK), jnp.int32),
        pltpu.VMEM((NCH, CHUNK), jnp.int32),
        pltpu.VMEM((NBUF, CHUNK, HF), jnp.float32),
        pltpu.VMEM((ZROWS, HF), jnp.float32),
        pltpu.VMEM_SHARED((NP, HF), jnp.float32),
        [pltpu.SemaphoreType.DMA] * NBUF,
        [pltpu.SemaphoreType.DMA] * NBUF,
    ],
)
def _sc_spmm(fn_hbm, src_hbm, dst_hbm, out_hbm,
             src_v, dst_v, rows_v, zbuf_v, acc_sh, gsem, ssem):
    c = lax.axis_index("c")
    s = lax.axis_index("s")
    wid = c * NS + s

    zero = jnp.zeros((16,), jnp.float32)

    def _zero_body(t, carry):
        zbuf_v[t // (HF // 16), pl.ds((t % (HF // 16)) * 16, 16)] = zero
        return carry

    lax.fori_loop(0, ZROWS * (HF // 16), _zero_body, 0)

    pltpu.sync_copy(src_hbm.at[wid], src_v)
    pltpu.sync_copy(dst_hbm.at[wid], dst_v)

    for h in range(2):
        for k in range(RPT // ZROWS):
            pltpu.sync_copy(zbuf_v, acc_sh.at[pl.ds(s * RPT + k * ZROWS, ZROWS)])
        plsc.subcore_barrier()

        def _group_body(g, carry):
            j0 = g * NBUF
            gd = [pltpu.async_copy(fn_hbm.at[h].at[src_v.at[j0 + b]],
                                   rows_v.at[b], gsem[b])
                  for b in range(NBUF)]
            sd = []
            for b in range(NBUF):
                gd[b].wait()
                sd.append(pltpu.async_copy(rows_v.at[b],
                                           acc_sh.at[dst_v.at[j0 + b]],
                                           ssem[b], add=True))
            for b in range(NBUF):
                sd[b].wait()
            return carry

        lax.fori_loop(0, NGRP, _group_body, 0)
        plsc.subcore_barrier()

        for k in range(RPT // ZROWS):
            base = s * RPT + k * ZROWS
            pltpu.sync_copy(acc_sh.at[pl.ds(base, ZROWS)],
                            out_hbm.at[c, h, pl.ds(base, ZROWS)])
        plsc.subcore_barrier()


def kernel(feat, edge_index, W_self, W_neigh):
    ei = edge_index.astype(jnp.int32)
    pad = EPWP - EPW
    pad_src = (jnp.arange(pad, dtype=jnp.int32) * 37) % N_NODES
    src = jnp.concatenate(
        [ei[0].reshape(NW, EPW), jnp.broadcast_to(pad_src, (NW, pad))],
        axis=1).reshape(NW, NCH, CHUNK)
    pad_dst = N_NODES + (jnp.arange(pad, dtype=jnp.int32) % (NP - N_NODES))
    dst = jnp.concatenate(
        [ei[1].reshape(NW, EPW), jnp.broadcast_to(pad_dst, (NW, pad))],
        axis=1).reshape(NW, NCH, CHUNK)
    fn = _mm(feat, W_neigh.reshape(2, HF, FEATS))
    parts = _sc_spmm(fn, src, dst)
    return _mm_add(feat, W_self, parts)
